# baseline (speedup 1.0000x reference)
import functools

import jax
import jax.numpy as jnp
from jax import lax
from jax.experimental import pallas as pl
from jax.experimental.pallas import tpu as pltpu
from jax.experimental.pallas import tpu_sc as plsc

L = 128
NN = 10000
NE = 320000
NSL = 2
H = NE // NSL
NC = 2
NS = 16
NW = NC * NS
EPW = H // NW
CHUNK = 40
NCH = EPW // CHUNK
NNP = 10240
NPT = NNP // NS

_F32 = jnp.float32
_BF16 = jnp.bfloat16


@functools.cache
def _sc_kernels():
    mesh = plsc.VectorSubcoreMesh(
        core_axis_name="c", subcore_axis_name="s",
        num_cores=NC, num_subcores=NS)

    @functools.partial(
        pl.kernel,
        out_type=[
            jax.ShapeDtypeStruct((H, L), _F32),
            jax.ShapeDtypeStruct((H, L), _F32),
        ],
        mesh=mesh,
        scratch_types=[
            pltpu.VMEM((NCH, CHUNK), jnp.int32),
            pltpu.VMEM((NCH, CHUNK), jnp.int32),
            pltpu.VMEM((CHUNK, L), _F32),
            pltpu.VMEM((CHUNK, L), _F32),
            pltpu.VMEM((8, L), _F32),
            pltpu.SemaphoreType.DMA,
            pltpu.SemaphoreType.DMA,
        ],
    )
    def _sc_gather2(p_hbm, q_hbm, di_hbm, si_hbm, tok_hbm, pg_hbm, qg_hbm,
                    idx_d, idx_s, bufp, bufq, tokbuf, semp, semq):
        wid = lax.axis_index("s") * NC + lax.axis_index("c")
        base = wid * EPW
        pltpu.sync_copy(tok_hbm, tokbuf)
        pltpu.sync_copy(di_hbm.at[wid], idx_d)
        pltpu.sync_copy(si_hbm.at[wid], idx_s)

        def body(j, carry):
            row = base + j * CHUNK
            cp = pltpu.async_copy(p_hbm.at[idx_d.at[j]], bufp, semp)
            cq = pltpu.async_copy(q_hbm.at[idx_s.at[j]], bufq, semq)
            cp.wait()
            pltpu.sync_copy(bufp, pg_hbm.at[pl.ds(row, CHUNK)])
            cq.wait()
            pltpu.sync_copy(bufq, qg_hbm.at[pl.ds(row, CHUNK)])
            return carry

        lax.fori_loop(0, NCH, body, 0)

    @functools.partial(
        pl.kernel,
        out_type=jax.ShapeDtypeStruct((2, NNP, L), _F32),
        mesh=mesh,
        scratch_types=[
            pltpu.VMEM((NCH, CHUNK), jnp.int32),
            pltpu.VMEM((CHUNK, L), _F32),
            pltpu.VMEM_SHARED((NNP, L), _F32),
        ],
    )
    def _sc_scatter(msg_hbm, di_hbm, zero_hbm, out_hbm, idx_d, buf, accum):
        cid = lax.axis_index("c")
        sid = lax.axis_index("s")
        wid = sid * NC + cid
        base = wid * EPW
        pltpu.sync_copy(di_hbm.at[wid], idx_d)
        r0 = sid * NPT
        pltpu.sync_copy(zero_hbm.at[pl.ds(r0, NPT)], accum.at[pl.ds(r0, NPT)])
        plsc.subcore_barrier()

        def body(j, carry):
            row = base + j * CHUNK
            pltpu.sync_copy(msg_hbm.at[pl.ds(row, CHUNK)], buf)
            pltpu.sync_copy(buf, accum.at[idx_d.at[j]], add=True)
            return carry

        lax.fori_loop(0, NCH, body, 0)
        plsc.subcore_barrier()
        pltpu.sync_copy(accum.at[pl.ds(r0, NPT)],
                        out_hbm.at[cid, pl.ds(r0, NPT)])

    return _sc_gather2, _sc_scatter


def _dot(a, b):
    return jnp.dot(a, b, preferred_element_type=_F32)


def _dotb(a, b):
    return jnp.dot(a.astype(_BF16), b.astype(_BF16),
                   preferred_element_type=_F32)


def _ln_block(m, g, b):
    mu = jnp.mean(m, axis=-1, keepdims=True)
    d = m - mu
    var = jnp.mean(d * d, axis=-1, keepdims=True)
    return d * lax.rsqrt(var + 1e-5) * g + b


def _prep_body(x, wa, wb, po, qo):
    xv = x[...]
    po[...] = _dot(xv, wa[...])
    qo[...] = _dot(xv, wb[...])


def _edge_body(pg, qg, ea, w1c, b1, w2, b2, w3, b3, w4, b4, g, bb,
               msg_o, ean_o):
    ea_v = ea[...]
    h = pg[...] + qg[...] + _dotb(ea_v, w1c[...]) + b1[...]
    h = jax.nn.relu(h)
    h = jax.nn.relu(_dotb(h, w2[...]) + b2[...])
    h = jax.nn.relu(_dotb(h, w3[...]) + b3[...])
    m = _dotb(h, w4[...]) + b4[...]
    msg = _ln_block(m, g[...], bb[...])
    msg_o[...] = msg
    ean_o[...] = ea_v + msg


def _node_body(x, p00, p01, p10, p11, w1a, w1b, b1, w2, b2, w3, b3, w4, b4,
               g, bb, *rest):
    xv = x[...]
    agg = (p00[...] + p01[...]) + (p10[...] + p11[...])
    h = _dotb(xv, w1a[...]) + _dotb(agg, w1b[...]) + b1[...]
    h = jax.nn.relu(h)
    h = jax.nn.relu(_dotb(h, w2[...]) + b2[...])
    h = jax.nn.relu(_dotb(h, w3[...]) + b3[...])
    u = _dotb(h, w4[...]) + b4[...]
    xn = xv + _ln_block(u, g[...], bb[...])
    if len(rest) == 1:
        (xo,) = rest
        xo[...] = xn
    else:
        wea, web, xo, po, qo = rest
        xo[...] = xn
        po[...] = _dot(xn, wea[...])
        qo[...] = _dot(xn, web[...])


_EB = 1280
_NB = 2000


def _bspec(rows):
    return pl.BlockSpec((rows, L), lambda i: (i, 0))


_WSPEC = pl.BlockSpec((L, L), lambda i: (0, 0))
_VSPEC = pl.BlockSpec((1, L), lambda i: (0, 0))


def _prep_tc(x, wa, wb):
    return pl.pallas_call(
        _prep_body,
        grid=(NN // _NB,),
        in_specs=[_bspec(_NB), _WSPEC, _WSPEC],
        out_specs=[_bspec(_NB), _bspec(_NB)],
        out_shape=[jax.ShapeDtypeStruct((NN, L), _F32)] * 2,
    )(x, wa, wb)


def _edge_tc(pg, qg, ea, w1c, b1, w2, b2, w3, b3, w4, b4, g, bb):
    blk = _bspec(_EB)
    return pl.pallas_call(
        _edge_body,
        grid=(H // _EB,),
        in_specs=[blk, blk, blk, _WSPEC, _VSPEC, _WSPEC, _VSPEC,
                  _WSPEC, _VSPEC, _WSPEC, _VSPEC, _VSPEC, _VSPEC],
        out_specs=[blk, blk],
        out_shape=[jax.ShapeDtypeStruct((H, L), _F32)] * 2,
    )(pg, qg, ea, w1c, b1, w2, b2, w3, b3, w4, b4, g, bb)


def _node_tc(x, parts, w1a, w1b, b1, w2, b2, w3, b3, w4, b4, g, bb,
             wea=None, web=None):
    blk = _bspec(_NB)
    n_out = 1 if wea is None else 3
    specs = [blk] * 5 + [_WSPEC, _WSPEC, _VSPEC, _WSPEC, _VSPEC,
                         _WSPEC, _VSPEC, _WSPEC, _VSPEC, _VSPEC, _VSPEC]
    args = [x] + parts + [w1a, w1b, b1, w2, b2, w3, b3, w4, b4, g, bb]
    if wea is not None:
        specs += [_WSPEC, _WSPEC]
        args += [wea, web]
    return pl.pallas_call(
        _node_body,
        grid=(NN // _NB,),
        in_specs=specs,
        out_specs=[blk] * n_out,
        out_shape=[jax.ShapeDtypeStruct((NN, L), _F32)] * n_out,
    )(*args)


def kernel(x, edge_attr, edge_index, We1, be1, We2, be2, We3, be3, We4, be4,
           ge, bbe, Wn1, bn1, Wn2, bn2, Wn3, bn3, Wn4, bn4, gn, bbn):
    src = edge_index[0]
    dst = edge_index[1]
    di3 = [dst[k * H:(k + 1) * H].reshape(NW, NCH, CHUNK) for k in range(NSL)]
    si3 = [src[k * H:(k + 1) * H].reshape(NW, NCH, CHUNK) for k in range(NSL)]
    zeros = jnp.zeros((NNP, L), _F32)

    eah = [edge_attr[k * H:(k + 1) * H] for k in range(NSL)]
    sc_gather2, sc_scatter = _sc_kernels()
    P, Q = _prep_tc(x, We1[0, :L], We1[0, L:2 * L])
    tok = jnp.zeros((8, L), _F32)
    for s in range(2):
        ew = (We1[s, 2 * L:], be1[s][None], We2[s], be2[s][None],
              We3[s], be3[s][None], We4[s], be4[s][None],
              ge[s][None], bbe[s][None])
        msgs = []
        for k in range(NSL):
            pg, qg = sc_gather2(P, Q, di3[k], si3[k], tok)
            tok = lax.slice(pg, (0, 0), (8, L))
            msg, ean = _edge_tc(pg, qg, eah[k], *ew)
            msgs.append(msg)
            eah[k] = ean
        parts = []
        for k in range(NSL):
            pk = sc_scatter(msgs[k], di3[k], zeros)
            parts += [pk[0], pk[1]]
        nw = (Wn1[s, :L], Wn1[s, L:], bn1[s][None], Wn2[s], bn2[s][None],
              Wn3[s], bn3[s][None], Wn4[s], bn4[s][None],
              gn[s][None], bbn[s][None])
        if s == 0:
            x, P, Q = _node_tc(x, parts, *nw,
                               wea=We1[1, :L], web=We1[1, L:2 * L])
        else:
            (x,) = _node_tc(x, parts, *nw)
    return (x, jnp.concatenate(eah, axis=0))


# speedup vs baseline: 1.1488x; 1.1488x over previous
import functools

import jax
import jax.numpy as jnp
from jax import lax
from jax.experimental import pallas as pl
from jax.experimental.pallas import tpu as pltpu
from jax.experimental.pallas import tpu_sc as plsc

L = 128
NN = 10000
NE = 320000
NC = 2
NS = 16
NW = NC * NS
EPW = NE // NW
CHUNK = 80
NCH = EPW // CHUNK
NNP = 10240
NPT = NNP // NS

_F32 = jnp.float32
_BF16 = jnp.bfloat16


@functools.cache
def _sc_kernels():
    mesh = plsc.VectorSubcoreMesh(
        core_axis_name="c", subcore_axis_name="s",
        num_cores=NC, num_subcores=NS)

    @functools.partial(
        pl.kernel,
        out_type=[
            jax.ShapeDtypeStruct((NE, L), _F32),
            jax.ShapeDtypeStruct((NE, L), _F32),
        ],
        mesh=mesh,
        scratch_types=[
            pltpu.VMEM((NCH, CHUNK), jnp.int32),
            pltpu.VMEM((NCH, CHUNK), jnp.int32),
            pltpu.VMEM((CHUNK, L), _F32),
            pltpu.VMEM((CHUNK, L), _F32),
            pltpu.SemaphoreType.DMA,
            pltpu.SemaphoreType.DMA,
        ],
    )
    def _sc_gather2(p_hbm, q_hbm, di_hbm, si_hbm, pg_hbm, qg_hbm,
                    idx_d, idx_s, bufp, bufq, semp, semq):
        wid = lax.axis_index("s") * NC + lax.axis_index("c")
        base = wid * EPW
        pltpu.sync_copy(di_hbm.at[wid], idx_d)
        pltpu.sync_copy(si_hbm.at[wid], idx_s)

        def body(j, carry):
            row = base + j * CHUNK
            cp = pltpu.async_copy(p_hbm.at[idx_d.at[j]], bufp, semp)
            cq = pltpu.async_copy(q_hbm.at[idx_s.at[j]], bufq, semq)
            cp.wait()
            pltpu.sync_copy(bufp, pg_hbm.at[pl.ds(row, CHUNK)])
            cq.wait()
            pltpu.sync_copy(bufq, qg_hbm.at[pl.ds(row, CHUNK)])
            return carry

        lax.fori_loop(0, NCH, body, 0)

    @functools.partial(
        pl.kernel,
        out_type=jax.ShapeDtypeStruct((2, NNP, L), _F32),
        mesh=mesh,
        scratch_types=[
            pltpu.VMEM((NCH, CHUNK), jnp.int32),
            pltpu.VMEM((CHUNK, L), _F32),
            pltpu.VMEM_SHARED((NNP, L), _F32),
        ],
    )
    def _sc_scatter(msg_hbm, di_hbm, zero_hbm, out_hbm, idx_d, buf, accum):
        cid = lax.axis_index("c")
        sid = lax.axis_index("s")
        wid = sid * NC + cid
        base = wid * EPW
        pltpu.sync_copy(di_hbm.at[wid], idx_d)
        r0 = sid * NPT
        pltpu.sync_copy(zero_hbm.at[pl.ds(r0, NPT)], accum.at[pl.ds(r0, NPT)])
        plsc.subcore_barrier()

        def body(j, carry):
            row = base + j * CHUNK
            pltpu.sync_copy(msg_hbm.at[pl.ds(row, CHUNK)], buf)
            pltpu.sync_copy(buf, accum.at[idx_d.at[j]], add=True)
            return carry

        lax.fori_loop(0, NCH, body, 0)
        plsc.subcore_barrier()
        pltpu.sync_copy(accum.at[pl.ds(r0, NPT)],
                        out_hbm.at[cid, pl.ds(r0, NPT)])

    return _sc_gather2, _sc_scatter


def _dot(a, b):
    return jnp.dot(a, b, preferred_element_type=_F32)


def _dotb(a, b):
    return jnp.dot(a.astype(_BF16), b.astype(_BF16),
                   preferred_element_type=_F32)


def _ln_block(m, g, b):
    mu = jnp.mean(m, axis=-1, keepdims=True)
    d = m - mu
    var = jnp.mean(d * d, axis=-1, keepdims=True)
    return d * lax.rsqrt(var + 1e-5) * g + b


def _prep_body(x, wa, wb, po, qo):
    xv = x[...]
    po[...] = _dot(xv, wa[...])
    qo[...] = _dot(xv, wb[...])


def _edge_body(pg, qg, ea, w1c, b1, w2, b2, w3, b3, w4, b4, g, bb,
               msg_o, ean_o):
    ea_v = ea[...]
    h = pg[...] + qg[...] + _dotb(ea_v, w1c[...]) + b1[...]
    h = jax.nn.relu(h)
    h = jax.nn.relu(_dotb(h, w2[...]) + b2[...])
    h = jax.nn.relu(_dotb(h, w3[...]) + b3[...])
    m = _dotb(h, w4[...]) + b4[...]
    msg = _ln_block(m, g[...], bb[...])
    msg_o[...] = msg
    ean_o[...] = ea_v + msg


def _node_body(x, p0, p1, w1a, w1b, b1, w2, b2, w3, b3, w4, b4,
               g, bb, *rest):
    xv = x[...]
    agg = p0[...] + p1[...]
    h = _dotb(xv, w1a[...]) + _dotb(agg, w1b[...]) + b1[...]
    h = jax.nn.relu(h)
    h = jax.nn.relu(_dotb(h, w2[...]) + b2[...])
    h = jax.nn.relu(_dotb(h, w3[...]) + b3[...])
    u = _dotb(h, w4[...]) + b4[...]
    xn = xv + _ln_block(u, g[...], bb[...])
    if len(rest) == 1:
        (xo,) = rest
        xo[...] = xn
    else:
        wea, web, xo, po, qo = rest
        xo[...] = xn
        po[...] = _dot(xn, wea[...])
        qo[...] = _dot(xn, web[...])


_EB = 2560
_NB = 2000


def _bspec(rows):
    return pl.BlockSpec((rows, L), lambda i: (i, 0))


_WSPEC = pl.BlockSpec((L, L), lambda i: (0, 0))
_VSPEC = pl.BlockSpec((1, L), lambda i: (0, 0))


def _prep_tc(x, wa, wb):
    return pl.pallas_call(
        _prep_body,
        grid=(NN // _NB,),
        in_specs=[_bspec(_NB), _WSPEC, _WSPEC],
        out_specs=[_bspec(_NB), _bspec(_NB)],
        out_shape=[jax.ShapeDtypeStruct((NN, L), _F32)] * 2,
    )(x, wa, wb)


def _edge_tc(pg, qg, ea, w1c, b1, w2, b2, w3, b3, w4, b4, g, bb):
    blk = _bspec(_EB)
    return pl.pallas_call(
        _edge_body,
        grid=(NE // _EB,),
        in_specs=[blk, blk, blk, _WSPEC, _VSPEC, _WSPEC, _VSPEC,
                  _WSPEC, _VSPEC, _WSPEC, _VSPEC, _VSPEC, _VSPEC],
        out_specs=[blk, blk],
        out_shape=[jax.ShapeDtypeStruct((NE, L), _F32)] * 2,
    )(pg, qg, ea, w1c, b1, w2, b2, w3, b3, w4, b4, g, bb)


def _node_tc(x, parts, w1a, w1b, b1, w2, b2, w3, b3, w4, b4, g, bb,
             wea=None, web=None):
    blk = _bspec(_NB)
    n_out = 1 if wea is None else 3
    specs = [blk] * 3 + [_WSPEC, _WSPEC, _VSPEC, _WSPEC, _VSPEC,
                         _WSPEC, _VSPEC, _WSPEC, _VSPEC, _VSPEC, _VSPEC]
    args = [x] + parts + [w1a, w1b, b1, w2, b2, w3, b3, w4, b4, g, bb]
    if wea is not None:
        specs += [_WSPEC, _WSPEC]
        args += [wea, web]
    return pl.pallas_call(
        _node_body,
        grid=(NN // _NB,),
        in_specs=specs,
        out_specs=[blk] * n_out,
        out_shape=[jax.ShapeDtypeStruct((NN, L), _F32)] * n_out,
    )(*args)


def kernel(x, edge_attr, edge_index, We1, be1, We2, be2, We3, be3, We4, be4,
           ge, bbe, Wn1, bn1, Wn2, bn2, Wn3, bn3, Wn4, bn4, gn, bbn):
    src = edge_index[0]
    dst = edge_index[1]
    di3 = dst.reshape(NW, NCH, CHUNK)
    si3 = src.reshape(NW, NCH, CHUNK)
    zeros = jnp.zeros((NNP, L), _F32)

    ea = edge_attr
    sc_gather2, sc_scatter = _sc_kernels()
    P, Q = _prep_tc(x, We1[0, :L], We1[0, L:2 * L])
    for s in range(2):
        ew = (We1[s, 2 * L:], be1[s][None], We2[s], be2[s][None],
              We3[s], be3[s][None], We4[s], be4[s][None],
              ge[s][None], bbe[s][None])
        pg, qg = sc_gather2(P, Q, di3, si3)
        msg, ea = _edge_tc(pg, qg, ea, *ew)
        pk = sc_scatter(msg, di3, zeros)
        nw = (Wn1[s, :L], Wn1[s, L:], bn1[s][None], Wn2[s], bn2[s][None],
              Wn3[s], bn3[s][None], Wn4[s], bn4[s][None],
              gn[s][None], bbn[s][None])
        if s == 0:
            x, P, Q = _node_tc(x, [pk[0], pk[1]], *nw,
                               wea=We1[1, :L], web=We1[1, L:2 * L])
        else:
            (x,) = _node_tc(x, [pk[0], pk[1]], *nw)
    return (x, ea)


# speedup vs baseline: 1.1905x; 1.0363x over previous
import functools

import jax
import jax.numpy as jnp
from jax import lax
from jax.experimental import pallas as pl
from jax.experimental.pallas import tpu as pltpu
from jax.experimental.pallas import tpu_sc as plsc

L = 128
NN = 10000
NE = 320000
NC = 2
NS = 16
NW = NC * NS
EPW = NE // NW
CHUNK = 80
NCH = EPW // CHUNK
NNP = 10240
NPT = NNP // NS

_F32 = jnp.float32
_BF16 = jnp.bfloat16


@functools.cache
def _sc_kernels():
    mesh = plsc.VectorSubcoreMesh(
        core_axis_name="c", subcore_axis_name="s",
        num_cores=NC, num_subcores=NS)

    @functools.partial(
        pl.kernel,
        out_type=[
            jax.ShapeDtypeStruct((NE, L), _F32),
            jax.ShapeDtypeStruct((NE, L), _F32),
        ],
        mesh=mesh,
        scratch_types=[
            pltpu.VMEM((NCH, CHUNK), jnp.int32),
            pltpu.VMEM((NCH, CHUNK), jnp.int32),
            pltpu.VMEM((CHUNK, L), _F32),
            pltpu.VMEM((CHUNK, L), _F32),
            pltpu.SemaphoreType.DMA,
            pltpu.SemaphoreType.DMA,
        ],
    )
    def _sc_gather2(p_hbm, q_hbm, di_hbm, si_hbm, pg_hbm, qg_hbm,
                    idx_d, idx_s, bufp, bufq, semp, semq):
        wid = lax.axis_index("s") * NC + lax.axis_index("c")
        base = wid * EPW
        pltpu.sync_copy(di_hbm.at[wid], idx_d)
        pltpu.sync_copy(si_hbm.at[wid], idx_s)

        def body(j, carry):
            row = base + j * CHUNK
            cp = pltpu.async_copy(p_hbm.at[idx_d.at[j]], bufp, semp)
            cq = pltpu.async_copy(q_hbm.at[idx_s.at[j]], bufq, semq)
            cp.wait()
            pltpu.sync_copy(bufp, pg_hbm.at[pl.ds(row, CHUNK)])
            cq.wait()
            pltpu.sync_copy(bufq, qg_hbm.at[pl.ds(row, CHUNK)])
            return carry

        lax.fori_loop(0, NCH, body, 0)

    @functools.partial(
        pl.kernel,
        out_type=jax.ShapeDtypeStruct((2, NNP, L), _F32),
        mesh=mesh,
        scratch_types=[
            pltpu.VMEM((NCH, CHUNK), jnp.int32),
            pltpu.VMEM((CHUNK, L), _F32),
            pltpu.VMEM_SHARED((NNP, L), _F32),
        ],
    )
    def _sc_scatter(msg_hbm, di_hbm, zero_hbm, out_hbm, idx_d, buf, accum):
        cid = lax.axis_index("c")
        sid = lax.axis_index("s")
        wid = sid * NC + cid
        base = wid * EPW
        pltpu.sync_copy(di_hbm.at[wid], idx_d)
        r0 = sid * NPT
        pltpu.sync_copy(zero_hbm.at[pl.ds(r0, NPT)], accum.at[pl.ds(r0, NPT)])
        plsc.subcore_barrier()

        def body(j, carry):
            row = base + j * CHUNK
            pltpu.sync_copy(msg_hbm.at[pl.ds(row, CHUNK)], buf)
            pltpu.sync_copy(buf, accum.at[idx_d.at[j]], add=True)
            return carry

        lax.fori_loop(0, NCH, body, 0)
        plsc.subcore_barrier()
        pltpu.sync_copy(accum.at[pl.ds(r0, NPT)],
                        out_hbm.at[cid, pl.ds(r0, NPT)])

    return _sc_gather2, _sc_scatter


def _dot(a, b):
    return jnp.dot(a, b, preferred_element_type=_F32)


def _dotb(a, b):
    return jnp.dot(a.astype(_BF16), b.astype(_BF16),
                   preferred_element_type=_F32)


def _ln_block(m, g, b):
    mu = jnp.mean(m, axis=-1, keepdims=True)
    d = m - mu
    var = jnp.mean(d * d, axis=-1, keepdims=True)
    return d * lax.rsqrt(var + 1e-5) * g + b


def _prep_body(x, wa, wb, po, qo):
    xv = x[...]
    po[...] = _dot(xv, wa[...])
    qo[...] = _dot(xv, wb[...])


def _edge_body(pg, qg, ea, w1c, b1, w2, b2, w3, b3, w4, b4, g, bb,
               msg_o, ean_o):
    ea_v = ea[...]
    h = pg[...] + qg[...] + _dotb(ea_v, w1c[...]) + b1[...]
    h = jax.nn.relu(h)
    h = jax.nn.relu(_dotb(h, w2[...]) + b2[...])
    h = jax.nn.relu(_dotb(h, w3[...]) + b3[...])
    m = _dotb(h, w4[...]) + b4[...]
    msg = _ln_block(m, g[...], bb[...])
    msg_o[...] = msg
    ean_o[...] = ea_v + msg


def _node_body(x, p0, p1, w1a, w1b, b1, w2, b2, w3, b3, w4, b4,
               g, bb, *rest):
    xv = x[...]
    agg = p0[...] + p1[...]
    h = _dotb(xv, w1a[...]) + _dotb(agg, w1b[...]) + b1[...]
    h = jax.nn.relu(h)
    h = jax.nn.relu(_dotb(h, w2[...]) + b2[...])
    h = jax.nn.relu(_dotb(h, w3[...]) + b3[...])
    u = _dotb(h, w4[...]) + b4[...]
    xn = xv + _ln_block(u, g[...], bb[...])
    if len(rest) == 1:
        (xo,) = rest
        xo[...] = xn
    else:
        wea, web, xo, po, qo = rest
        xo[...] = xn
        po[...] = _dot(xn, wea[...])
        qo[...] = _dot(xn, web[...])


_EB = 4000
_NB = 2000


def _bspec(rows):
    return pl.BlockSpec((rows, L), lambda i: (i, 0))


_WSPEC = pl.BlockSpec((L, L), lambda i: (0, 0))
_VSPEC = pl.BlockSpec((1, L), lambda i: (0, 0))


def _prep_tc(x, wa, wb):
    return pl.pallas_call(
        _prep_body,
        grid=(NN // _NB,),
        in_specs=[_bspec(_NB), _WSPEC, _WSPEC],
        out_specs=[_bspec(_NB), _bspec(_NB)],
        out_shape=[jax.ShapeDtypeStruct((NN, L), _F32)] * 2,
    )(x, wa, wb)


def _edge_tc(pg, qg, ea, w1c, b1, w2, b2, w3, b3, w4, b4, g, bb):
    blk = _bspec(_EB)
    return pl.pallas_call(
        _edge_body,
        grid=(NE // _EB,),
        in_specs=[blk, blk, blk, _WSPEC, _VSPEC, _WSPEC, _VSPEC,
                  _WSPEC, _VSPEC, _WSPEC, _VSPEC, _VSPEC, _VSPEC],
        out_specs=[blk, blk],
        out_shape=[jax.ShapeDtypeStruct((NE, L), _F32)] * 2,
    )(pg, qg, ea, w1c, b1, w2, b2, w3, b3, w4, b4, g, bb)


def _node_tc(x, parts, w1a, w1b, b1, w2, b2, w3, b3, w4, b4, g, bb,
             wea=None, web=None):
    blk = _bspec(_NB)
    n_out = 1 if wea is None else 3
    specs = [blk] * 3 + [_WSPEC, _WSPEC, _VSPEC, _WSPEC, _VSPEC,
                         _WSPEC, _VSPEC, _WSPEC, _VSPEC, _VSPEC, _VSPEC]
    args = [x] + parts + [w1a, w1b, b1, w2, b2, w3, b3, w4, b4, g, bb]
    if wea is not None:
        specs += [_WSPEC, _WSPEC]
        args += [wea, web]
    return pl.pallas_call(
        _node_body,
        grid=(NN // _NB,),
        in_specs=specs,
        out_specs=[blk] * n_out,
        out_shape=[jax.ShapeDtypeStruct((NN, L), _F32)] * n_out,
    )(*args)


def kernel(x, edge_attr, edge_index, We1, be1, We2, be2, We3, be3, We4, be4,
           ge, bbe, Wn1, bn1, Wn2, bn2, Wn3, bn3, Wn4, bn4, gn, bbn):
    src = edge_index[0]
    dst = edge_index[1]
    di3 = dst.reshape(NW, NCH, CHUNK)
    si3 = src.reshape(NW, NCH, CHUNK)
    zeros = jnp.zeros((NNP, L), _F32)

    ea = edge_attr
    sc_gather2, sc_scatter = _sc_kernels()
    P, Q = _prep_tc(x, We1[0, :L], We1[0, L:2 * L])
    for s in range(2):
        ew = (We1[s, 2 * L:], be1[s][None], We2[s], be2[s][None],
              We3[s], be3[s][None], We4[s], be4[s][None],
              ge[s][None], bbe[s][None])
        pg, qg = sc_gather2(P, Q, di3, si3)
        msg, ea = _edge_tc(pg, qg, ea, *ew)
        pk = sc_scatter(msg, di3, zeros)
        nw = (Wn1[s, :L], Wn1[s, L:], bn1[s][None], Wn2[s], bn2[s][None],
              Wn3[s], bn3[s][None], Wn4[s], bn4[s][None],
              gn[s][None], bbn[s][None])
        if s == 0:
            x, P, Q = _node_tc(x, [pk[0], pk[1]], *nw,
                               wea=We1[1, :L], web=We1[1, L:2 * L])
        else:
            (x,) = _node_tc(x, [pk[0], pk[1]], *nw)
    return (x, ea)


# speedup vs baseline: 1.1964x; 1.0050x over previous
import functools

import jax
import jax.numpy as jnp
from jax import lax
from jax.experimental import pallas as pl
from jax.experimental.pallas import tpu as pltpu
from jax.experimental.pallas import tpu_sc as plsc

L = 128
NN = 10000
NE = 320000
NC = 2
NS = 16
NW = NC * NS
EPW = NE // NW
CHUNK = 80
NCH = EPW // CHUNK
NNP = 10240
NPT = NNP // NS

_F32 = jnp.float32
_BF16 = jnp.bfloat16


@functools.cache
def _sc_kernels():
    mesh = plsc.VectorSubcoreMesh(
        core_axis_name="c", subcore_axis_name="s",
        num_cores=NC, num_subcores=NS)

    @functools.partial(
        pl.kernel,
        out_type=[
            jax.ShapeDtypeStruct((NE, L), _F32),
            jax.ShapeDtypeStruct((NE, L), _F32),
        ],
        mesh=mesh,
        scratch_types=[
            pltpu.VMEM((NCH, CHUNK), jnp.int32),
            pltpu.VMEM((NCH, CHUNK), jnp.int32),
            pltpu.VMEM((CHUNK, L), _F32),
            pltpu.VMEM((CHUNK, L), _F32),
            pltpu.SemaphoreType.DMA,
            pltpu.SemaphoreType.DMA,
        ],
    )
    def _sc_gather2(p_hbm, q_hbm, di_hbm, si_hbm, pg_hbm, qg_hbm,
                    idx_d, idx_s, bufp, bufq, semp, semq):
        wid = lax.axis_index("s") * NC + lax.axis_index("c")
        base = wid * EPW
        pltpu.sync_copy(di_hbm.at[wid], idx_d)
        pltpu.sync_copy(si_hbm.at[wid], idx_s)

        def body(j, carry):
            row = base + j * CHUNK
            cp = pltpu.async_copy(p_hbm.at[idx_d.at[j]], bufp, semp)
            cq = pltpu.async_copy(q_hbm.at[idx_s.at[j]], bufq, semq)
            cp.wait()
            pltpu.sync_copy(bufp, pg_hbm.at[pl.ds(row, CHUNK)])
            cq.wait()
            pltpu.sync_copy(bufq, qg_hbm.at[pl.ds(row, CHUNK)])
            return carry

        lax.fori_loop(0, NCH, body, 0)

    @functools.partial(
        pl.kernel,
        out_type=jax.ShapeDtypeStruct((2, NNP, L), _F32),
        mesh=mesh,
        scratch_types=[
            pltpu.VMEM((NCH, CHUNK), jnp.int32),
            pltpu.VMEM((CHUNK, L), _F32),
            pltpu.VMEM_SHARED((NNP, L), _F32),
        ],
    )
    def _sc_scatter(msg_hbm, di_hbm, zero_hbm, out_hbm, idx_d, buf, accum):
        cid = lax.axis_index("c")
        sid = lax.axis_index("s")
        wid = sid * NC + cid
        base = wid * EPW
        pltpu.sync_copy(di_hbm.at[wid], idx_d)
        r0 = sid * NPT
        pltpu.sync_copy(zero_hbm.at[pl.ds(r0, NPT)], accum.at[pl.ds(r0, NPT)])
        plsc.subcore_barrier()

        def body(j, carry):
            row = base + j * CHUNK
            pltpu.sync_copy(msg_hbm.at[pl.ds(row, CHUNK)], buf)
            pltpu.sync_copy(buf, accum.at[idx_d.at[j]], add=True)
            return carry

        lax.fori_loop(0, NCH, body, 0)
        plsc.subcore_barrier()
        pltpu.sync_copy(accum.at[pl.ds(r0, NPT)],
                        out_hbm.at[cid, pl.ds(r0, NPT)])

    return _sc_gather2, _sc_scatter


def _dot(a, b):
    return jnp.dot(a, b, preferred_element_type=_F32)


def _dotb(a, b):
    return jnp.dot(a.astype(_BF16), b.astype(_BF16),
                   preferred_element_type=_F32)


def _ln_block(m, g, b):
    mu = jnp.mean(m, axis=-1, keepdims=True)
    d = m - mu
    var = jnp.mean(d * d, axis=-1, keepdims=True)
    return d * lax.rsqrt(var + 1e-5) * g + b


def _prep_body(x, wa, wb, po, qo):
    xv = x[...]
    po[...] = _dot(xv, wa[...])
    qo[...] = _dot(xv, wb[...])


def _edge_body(pg, qg, ea, w1c, b1, w2, b2, w3, b3, w4, b4, g, bb,
               msg_o, ean_o):
    ea_v = ea[...]
    h = pg[...] + qg[...] + _dotb(ea_v, w1c[...]) + b1[...]
    h = jax.nn.relu(h)
    h = jax.nn.relu(_dotb(h, w2[...]) + b2[...])
    h = jax.nn.relu(_dotb(h, w3[...]) + b3[...])
    m = _dotb(h, w4[...]) + b4[...]
    msg = _ln_block(m, g[...], bb[...])
    msg_o[...] = msg
    ean_o[...] = ea_v + msg


def _node_body(x, p0, p1, w1a, w1b, b1, w2, b2, w3, b3, w4, b4,
               g, bb, *rest):
    xv = x[...]
    agg = p0[...] + p1[...]
    h = _dotb(xv, w1a[...]) + _dotb(agg, w1b[...]) + b1[...]
    h = jax.nn.relu(h)
    h = jax.nn.relu(_dotb(h, w2[...]) + b2[...])
    h = jax.nn.relu(_dotb(h, w3[...]) + b3[...])
    u = _dotb(h, w4[...]) + b4[...]
    xn = xv + _ln_block(u, g[...], bb[...])
    if len(rest) == 1:
        (xo,) = rest
        xo[...] = xn
    else:
        wea, web, xo, po, qo = rest
        xo[...] = xn
        po[...] = _dot(xn, wea[...])
        qo[...] = _dot(xn, web[...])


_EB = 8000
_NB = 2000


def _bspec(rows):
    return pl.BlockSpec((rows, L), lambda i: (i, 0))


_WSPEC = pl.BlockSpec((L, L), lambda i: (0, 0))
_VSPEC = pl.BlockSpec((1, L), lambda i: (0, 0))


def _prep_tc(x, wa, wb):
    return pl.pallas_call(
        _prep_body,
        grid=(NN // _NB,),
        in_specs=[_bspec(_NB), _WSPEC, _WSPEC],
        out_specs=[_bspec(_NB), _bspec(_NB)],
        out_shape=[jax.ShapeDtypeStruct((NN, L), _F32)] * 2,
    )(x, wa, wb)


def _edge_tc(pg, qg, ea, w1c, b1, w2, b2, w3, b3, w4, b4, g, bb):
    blk = _bspec(_EB)
    return pl.pallas_call(
        _edge_body,
        grid=(NE // _EB,),
        in_specs=[blk, blk, blk, _WSPEC, _VSPEC, _WSPEC, _VSPEC,
                  _WSPEC, _VSPEC, _WSPEC, _VSPEC, _VSPEC, _VSPEC],
        out_specs=[blk, blk],
        out_shape=[jax.ShapeDtypeStruct((NE, L), _F32)] * 2,
    )(pg, qg, ea, w1c, b1, w2, b2, w3, b3, w4, b4, g, bb)


def _node_tc(x, parts, w1a, w1b, b1, w2, b2, w3, b3, w4, b4, g, bb,
             wea=None, web=None):
    blk = _bspec(_NB)
    n_out = 1 if wea is None else 3
    specs = [blk] * 3 + [_WSPEC, _WSPEC, _VSPEC, _WSPEC, _VSPEC,
                         _WSPEC, _VSPEC, _WSPEC, _VSPEC, _VSPEC, _VSPEC]
    args = [x] + parts + [w1a, w1b, b1, w2, b2, w3, b3, w4, b4, g, bb]
    if wea is not None:
        specs += [_WSPEC, _WSPEC]
        args += [wea, web]
    return pl.pallas_call(
        _node_body,
        grid=(NN // _NB,),
        in_specs=specs,
        out_specs=[blk] * n_out,
        out_shape=[jax.ShapeDtypeStruct((NN, L), _F32)] * n_out,
    )(*args)


def kernel(x, edge_attr, edge_index, We1, be1, We2, be2, We3, be3, We4, be4,
           ge, bbe, Wn1, bn1, Wn2, bn2, Wn3, bn3, Wn4, bn4, gn, bbn):
    src = edge_index[0]
    dst = edge_index[1]
    di3 = dst.reshape(NW, NCH, CHUNK)
    si3 = src.reshape(NW, NCH, CHUNK)
    zeros = jnp.zeros((NNP, L), _F32)

    ea = edge_attr
    sc_gather2, sc_scatter = _sc_kernels()
    P, Q = _prep_tc(x, We1[0, :L], We1[0, L:2 * L])
    for s in range(2):
        ew = (We1[s, 2 * L:], be1[s][None], We2[s], be2[s][None],
              We3[s], be3[s][None], We4[s], be4[s][None],
              ge[s][None], bbe[s][None])
        pg, qg = sc_gather2(P, Q, di3, si3)
        msg, ea = _edge_tc(pg, qg, ea, *ew)
        pk = sc_scatter(msg, di3, zeros)
        nw = (Wn1[s, :L], Wn1[s, L:], bn1[s][None], Wn2[s], bn2[s][None],
              Wn3[s], bn3[s][None], Wn4[s], bn4[s][None],
              gn[s][None], bbn[s][None])
        if s == 0:
            x, P, Q = _node_tc(x, [pk[0], pk[1]], *nw,
                               wea=We1[1, :L], web=We1[1, L:2 * L])
        else:
            (x,) = _node_tc(x, [pk[0], pk[1]], *nw)
    return (x, ea)


# speedup vs baseline: 1.3398x; 1.1198x over previous
import functools

import jax
import jax.numpy as jnp
from jax import lax
from jax.experimental import pallas as pl
from jax.experimental.pallas import tpu as pltpu
from jax.experimental.pallas import tpu_sc as plsc

L = 128
NN = 10000
NE = 320000
NC = 2
NS = 16
NW = NC * NS
EPW = NE // NW
CHUNK = 40
NBUF = 4
NBS = 2
NCH = EPW // CHUNK
NNP = 10240
NPT = NNP // NS

_F32 = jnp.float32
_BF16 = jnp.bfloat16


@functools.cache
def _sc_kernels():
    mesh = plsc.VectorSubcoreMesh(
        core_axis_name="c", subcore_axis_name="s",
        num_cores=NC, num_subcores=NS)

    @functools.partial(
        pl.kernel,
        out_type=[
            jax.ShapeDtypeStruct((NE, L), _F32),
            jax.ShapeDtypeStruct((NE, L), _F32),
        ],
        mesh=mesh,
        scratch_types=[
            pltpu.VMEM((NCH, CHUNK), jnp.int32),
            pltpu.VMEM((NCH, CHUNK), jnp.int32),
            pltpu.VMEM((NBUF, CHUNK, L), _F32),
            pltpu.VMEM((NBUF, CHUNK, L), _F32),
            [pltpu.SemaphoreType.DMA] * NBUF,
            [pltpu.SemaphoreType.DMA] * NBUF,
            [pltpu.SemaphoreType.DMA] * NBUF,
            [pltpu.SemaphoreType.DMA] * NBUF,
        ],
    )
    def _sc_gather2(p_hbm, q_hbm, di_hbm, si_hbm, pg_hbm, qg_hbm,
                    idx_d, idx_s, bufp, bufq, sgp, sgq, swp, swq):
        wid = lax.axis_index("s") * NC + lax.axis_index("c")
        base = wid * EPW
        pltpu.sync_copy(di_hbm.at[wid], idx_d)
        pltpu.sync_copy(si_hbm.at[wid], idx_s)

        for b in range(NBUF):
            pltpu.async_copy(p_hbm.at[idx_d.at[b]], bufp.at[b], sgp[b])
            pltpu.async_copy(q_hbm.at[idx_s.at[b]], bufq.at[b], sgq[b])

        def body(g, carry):
            for b in range(NBUF):
                j = g * NBUF + b

                @pl.when(j < NCH)
                def _():
                    row = base + j * CHUNK
                    pltpu.make_async_copy(
                        p_hbm.at[idx_d.at[j]], bufp.at[b], sgp[b]).wait()
                    pltpu.async_copy(
                        bufp.at[b], pg_hbm.at[pl.ds(row, CHUNK)], swp[b])
                    pltpu.make_async_copy(
                        q_hbm.at[idx_s.at[j]], bufq.at[b], sgq[b]).wait()
                    pltpu.async_copy(
                        bufq.at[b], qg_hbm.at[pl.ds(row, CHUNK)], swq[b])
            for b in range(NBUF):
                jn = (g + 1) * NBUF + b

                @pl.when(jn < NCH)
                def _():
                    row = base + (jn - NBUF) * CHUNK
                    pltpu.make_async_copy(
                        bufp.at[b], pg_hbm.at[pl.ds(row, CHUNK)],
                        swp[b]).wait()
                    pltpu.async_copy(p_hbm.at[idx_d.at[jn]], bufp.at[b],
                                     sgp[b])
                    pltpu.make_async_copy(
                        bufq.at[b], qg_hbm.at[pl.ds(row, CHUNK)],
                        swq[b]).wait()
                    pltpu.async_copy(q_hbm.at[idx_s.at[jn]], bufq.at[b],
                                     sgq[b])
            return carry

        lax.fori_loop(0, (NCH + NBUF - 1) // NBUF, body, 0)
        for b in range(NBUF):
            row = base + (NCH - NBUF + ((b - NCH) % NBUF)) * CHUNK
            pltpu.make_async_copy(
                bufp.at[b], pg_hbm.at[pl.ds(row, CHUNK)], swp[b]).wait()
            pltpu.make_async_copy(
                bufq.at[b], qg_hbm.at[pl.ds(row, CHUNK)], swq[b]).wait()

    @functools.partial(
        pl.kernel,
        out_type=jax.ShapeDtypeStruct((2, NNP, L), _F32),
        mesh=mesh,
        scratch_types=[
            pltpu.VMEM((NCH, CHUNK), jnp.int32),
            pltpu.VMEM((NBS, CHUNK, L), _F32),
            pltpu.VMEM_SHARED((NNP, L), _F32),
            [pltpu.SemaphoreType.DMA] * NBS,
            [pltpu.SemaphoreType.DMA] * NBS,
        ],
    )
    def _sc_scatter(msg_hbm, di_hbm, zero_hbm, out_hbm, idx_d, buf, accum,
                    sr, ss):
        cid = lax.axis_index("c")
        sid = lax.axis_index("s")
        wid = sid * NC + cid
        base = wid * EPW
        pltpu.sync_copy(di_hbm.at[wid], idx_d)
        r0 = sid * NPT
        pltpu.sync_copy(zero_hbm.at[pl.ds(r0, NPT)], accum.at[pl.ds(r0, NPT)])
        plsc.subcore_barrier()

        for b in range(NBS):
            pltpu.async_copy(msg_hbm.at[pl.ds(base + b * CHUNK, CHUNK)],
                             buf.at[b], sr[b])

        def body(g, carry):
            for b in range(NBS):
                j = g * NBS + b

                @pl.when(j < NCH)
                def _():
                    row = base + j * CHUNK
                    pltpu.make_async_copy(
                        msg_hbm.at[pl.ds(row, CHUNK)], buf.at[b],
                        sr[b]).wait()
                    pltpu.async_copy(buf.at[b], accum.at[idx_d.at[j]], ss[b],
                                     add=True)
            for b in range(NBS):
                jn = (g + 1) * NBS + b

                @pl.when(jn < NCH)
                def _():
                    pltpu.make_async_copy(
                        buf.at[b], accum.at[idx_d.at[jn - NBS]],
                        ss[b]).wait()
                    pltpu.async_copy(
                        msg_hbm.at[pl.ds(base + jn * CHUNK, CHUNK)],
                        buf.at[b], sr[b])
            return carry

        lax.fori_loop(0, (NCH + NBS - 1) // NBS, body, 0)
        for b in range(NBS):
            pltpu.make_async_copy(buf.at[b], accum.at[idx_d.at[0]],
                                  ss[b]).wait()
        plsc.subcore_barrier()
        pltpu.sync_copy(accum.at[pl.ds(r0, NPT)],
                        out_hbm.at[cid, pl.ds(r0, NPT)])

    return _sc_gather2, _sc_scatter


def _dot(a, b):
    return jnp.dot(a, b, preferred_element_type=_F32)


def _dotb(a, b):
    return jnp.dot(a.astype(_BF16), b.astype(_BF16),
                   preferred_element_type=_F32)


def _ln_block(m, g, b):
    mu = jnp.mean(m, axis=-1, keepdims=True)
    d = m - mu
    var = jnp.mean(d * d, axis=-1, keepdims=True)
    return d * lax.rsqrt(var + 1e-5) * g + b


def _prep_body(x, wa, wb, po, qo):
    xv = x[...]
    po[...] = _dot(xv, wa[...])
    qo[...] = _dot(xv, wb[...])


def _edge_body(pg, qg, ea, w1c, b1, w2, b2, w3, b3, w4, b4, g, bb,
               msg_o, ean_o):
    ea_v = ea[...]
    h = pg[...] + qg[...] + _dotb(ea_v, w1c[...]) + b1[...]
    h = jax.nn.relu(h)
    h = jax.nn.relu(_dotb(h, w2[...]) + b2[...])
    h = jax.nn.relu(_dotb(h, w3[...]) + b3[...])
    m = _dotb(h, w4[...]) + b4[...]
    msg = _ln_block(m, g[...], bb[...])
    msg_o[...] = msg
    ean_o[...] = ea_v + msg


def _node_body(x, p0, p1, w1a, w1b, b1, w2, b2, w3, b3, w4, b4,
               g, bb, *rest):
    xv = x[...]
    agg = p0[...] + p1[...]
    h = _dotb(xv, w1a[...]) + _dotb(agg, w1b[...]) + b1[...]
    h = jax.nn.relu(h)
    h = jax.nn.relu(_dotb(h, w2[...]) + b2[...])
    h = jax.nn.relu(_dotb(h, w3[...]) + b3[...])
    u = _dotb(h, w4[...]) + b4[...]
    xn = xv + _ln_block(u, g[...], bb[...])
    if len(rest) == 1:
        (xo,) = rest
        xo[...] = xn
    else:
        wea, web, xo, po, qo = rest
        xo[...] = xn
        po[...] = _dot(xn, wea[...])
        qo[...] = _dot(xn, web[...])


_EB = 8000
_NB = 2000


def _bspec(rows):
    return pl.BlockSpec((rows, L), lambda i: (i, 0))


_WSPEC = pl.BlockSpec((L, L), lambda i: (0, 0))
_VSPEC = pl.BlockSpec((1, L), lambda i: (0, 0))


def _prep_tc(x, wa, wb):
    return pl.pallas_call(
        _prep_body,
        grid=(NN // _NB,),
        in_specs=[_bspec(_NB), _WSPEC, _WSPEC],
        out_specs=[_bspec(_NB), _bspec(_NB)],
        out_shape=[jax.ShapeDtypeStruct((NN, L), _F32)] * 2,
    )(x, wa, wb)


def _edge_tc(pg, qg, ea, w1c, b1, w2, b2, w3, b3, w4, b4, g, bb):
    blk = _bspec(_EB)
    return pl.pallas_call(
        _edge_body,
        grid=(NE // _EB,),
        in_specs=[blk, blk, blk, _WSPEC, _VSPEC, _WSPEC, _VSPEC,
                  _WSPEC, _VSPEC, _WSPEC, _VSPEC, _VSPEC, _VSPEC],
        out_specs=[blk, blk],
        out_shape=[jax.ShapeDtypeStruct((NE, L), _F32)] * 2,
    )(pg, qg, ea, w1c, b1, w2, b2, w3, b3, w4, b4, g, bb)


def _node_tc(x, parts, w1a, w1b, b1, w2, b2, w3, b3, w4, b4, g, bb,
             wea=None, web=None):
    blk = _bspec(_NB)
    n_out = 1 if wea is None else 3
    specs = [blk] * 3 + [_WSPEC, _WSPEC, _VSPEC, _WSPEC, _VSPEC,
                         _WSPEC, _VSPEC, _WSPEC, _VSPEC, _VSPEC, _VSPEC]
    args = [x] + parts + [w1a, w1b, b1, w2, b2, w3, b3, w4, b4, g, bb]
    if wea is not None:
        specs += [_WSPEC, _WSPEC]
        args += [wea, web]
    return pl.pallas_call(
        _node_body,
        grid=(NN // _NB,),
        in_specs=specs,
        out_specs=[blk] * n_out,
        out_shape=[jax.ShapeDtypeStruct((NN, L), _F32)] * n_out,
    )(*args)


def kernel(x, edge_attr, edge_index, We1, be1, We2, be2, We3, be3, We4, be4,
           ge, bbe, Wn1, bn1, Wn2, bn2, Wn3, bn3, Wn4, bn4, gn, bbn):
    src = edge_index[0]
    dst = edge_index[1]
    di3 = dst.reshape(NW, NCH, CHUNK)
    si3 = src.reshape(NW, NCH, CHUNK)
    zeros = jnp.zeros((NNP, L), _F32)

    ea = edge_attr
    sc_gather2, sc_scatter = _sc_kernels()
    P, Q = _prep_tc(x, We1[0, :L], We1[0, L:2 * L])
    for s in range(2):
        ew = (We1[s, 2 * L:], be1[s][None], We2[s], be2[s][None],
              We3[s], be3[s][None], We4[s], be4[s][None],
              ge[s][None], bbe[s][None])
        pg, qg = sc_gather2(P, Q, di3, si3)
        msg, ea = _edge_tc(pg, qg, ea, *ew)
        pk = sc_scatter(msg, di3, zeros)
        nw = (Wn1[s, :L], Wn1[s, L:], bn1[s][None], Wn2[s], bn2[s][None],
              Wn3[s], bn3[s][None], Wn4[s], bn4[s][None],
              gn[s][None], bbn[s][None])
        if s == 0:
            x, P, Q = _node_tc(x, [pk[0], pk[1]], *nw,
                               wea=We1[1, :L], web=We1[1, L:2 * L])
        else:
            (x,) = _node_tc(x, [pk[0], pk[1]], *nw)
    return (x, ea)


# speedup vs baseline: 1.3941x; 1.0405x over previous
import functools

import jax
import jax.numpy as jnp
from jax import lax
from jax.experimental import pallas as pl
from jax.experimental.pallas import tpu as pltpu
from jax.experimental.pallas import tpu_sc as plsc

L = 128
NN = 10000
NE = 320000
NC = 2
NS = 16
NW = NC * NS
EPW = NE // NW
CHUNK = 80
NBUF = 4
NBS = 2
NCH = EPW // CHUNK
NNP = 10240
NPT = NNP // NS

_F32 = jnp.float32
_BF16 = jnp.bfloat16


@functools.cache
def _sc_kernels():
    mesh = plsc.VectorSubcoreMesh(
        core_axis_name="c", subcore_axis_name="s",
        num_cores=NC, num_subcores=NS)

    @functools.partial(
        pl.kernel,
        out_type=[
            jax.ShapeDtypeStruct((NE, L), _F32),
            jax.ShapeDtypeStruct((NE, L), _F32),
        ],
        mesh=mesh,
        scratch_types=[
            pltpu.VMEM((NCH, CHUNK), jnp.int32),
            pltpu.VMEM((NCH, CHUNK), jnp.int32),
            pltpu.VMEM((NBUF, CHUNK, L), _F32),
            pltpu.VMEM((NBUF, CHUNK, L), _F32),
            [pltpu.SemaphoreType.DMA] * NBUF,
            [pltpu.SemaphoreType.DMA] * NBUF,
            [pltpu.SemaphoreType.DMA] * NBUF,
            [pltpu.SemaphoreType.DMA] * NBUF,
        ],
    )
    def _sc_gather2(p_hbm, q_hbm, di_hbm, si_hbm, pg_hbm, qg_hbm,
                    idx_d, idx_s, bufp, bufq, sgp, sgq, swp, swq):
        wid = lax.axis_index("s") * NC + lax.axis_index("c")
        base = wid * EPW
        pltpu.sync_copy(di_hbm.at[wid], idx_d)
        pltpu.sync_copy(si_hbm.at[wid], idx_s)

        for b in range(NBUF):
            pltpu.async_copy(p_hbm.at[idx_d.at[b]], bufp.at[b], sgp[b])
            pltpu.async_copy(q_hbm.at[idx_s.at[b]], bufq.at[b], sgq[b])

        def body(g, carry):
            for b in range(NBUF):
                j = g * NBUF + b

                @pl.when(j < NCH)
                def _():
                    row = base + j * CHUNK
                    pltpu.make_async_copy(
                        p_hbm.at[idx_d.at[j]], bufp.at[b], sgp[b]).wait()
                    pltpu.async_copy(
                        bufp.at[b], pg_hbm.at[pl.ds(row, CHUNK)], swp[b])
                    pltpu.make_async_copy(
                        q_hbm.at[idx_s.at[j]], bufq.at[b], sgq[b]).wait()
                    pltpu.async_copy(
                        bufq.at[b], qg_hbm.at[pl.ds(row, CHUNK)], swq[b])
            for b in range(NBUF):
                jn = (g + 1) * NBUF + b

                @pl.when(jn < NCH)
                def _():
                    row = base + (jn - NBUF) * CHUNK
                    pltpu.make_async_copy(
                        bufp.at[b], pg_hbm.at[pl.ds(row, CHUNK)],
                        swp[b]).wait()
                    pltpu.async_copy(p_hbm.at[idx_d.at[jn]], bufp.at[b],
                                     sgp[b])
                    pltpu.make_async_copy(
                        bufq.at[b], qg_hbm.at[pl.ds(row, CHUNK)],
                        swq[b]).wait()
                    pltpu.async_copy(q_hbm.at[idx_s.at[jn]], bufq.at[b],
                                     sgq[b])
            return carry

        lax.fori_loop(0, (NCH + NBUF - 1) // NBUF, body, 0)
        for b in range(NBUF):
            row = base + (NCH - NBUF + ((b - NCH) % NBUF)) * CHUNK
            pltpu.make_async_copy(
                bufp.at[b], pg_hbm.at[pl.ds(row, CHUNK)], swp[b]).wait()
            pltpu.make_async_copy(
                bufq.at[b], qg_hbm.at[pl.ds(row, CHUNK)], swq[b]).wait()

    @functools.partial(
        pl.kernel,
        out_type=jax.ShapeDtypeStruct((2, NNP, L), _F32),
        mesh=mesh,
        scratch_types=[
            pltpu.VMEM((NCH, CHUNK), jnp.int32),
            pltpu.VMEM((NBS, CHUNK, L), _F32),
            pltpu.VMEM_SHARED((NNP, L), _F32),
            [pltpu.SemaphoreType.DMA] * NBS,
            [pltpu.SemaphoreType.DMA] * NBS,
        ],
    )
    def _sc_scatter(msg_hbm, di_hbm, zero_hbm, out_hbm, idx_d, buf, accum,
                    sr, ss):
        cid = lax.axis_index("c")
        sid = lax.axis_index("s")
        wid = sid * NC + cid
        base = wid * EPW
        pltpu.sync_copy(di_hbm.at[wid], idx_d)
        r0 = sid * NPT
        pltpu.sync_copy(zero_hbm.at[pl.ds(r0, NPT)], accum.at[pl.ds(r0, NPT)])
        plsc.subcore_barrier()

        for b in range(NBS):
            pltpu.async_copy(msg_hbm.at[pl.ds(base + b * CHUNK, CHUNK)],
                             buf.at[b], sr[b])

        def body(g, carry):
            for b in range(NBS):
                j = g * NBS + b

                @pl.when(j < NCH)
                def _():
                    row = base + j * CHUNK
                    pltpu.make_async_copy(
                        msg_hbm.at[pl.ds(row, CHUNK)], buf.at[b],
                        sr[b]).wait()
                    pltpu.async_copy(buf.at[b], accum.at[idx_d.at[j]], ss[b],
                                     add=True)
            for b in range(NBS):
                jn = (g + 1) * NBS + b

                @pl.when(jn < NCH)
                def _():
                    pltpu.make_async_copy(
                        buf.at[b], accum.at[idx_d.at[jn - NBS]],
                        ss[b]).wait()
                    pltpu.async_copy(
                        msg_hbm.at[pl.ds(base + jn * CHUNK, CHUNK)],
                        buf.at[b], sr[b])
            return carry

        lax.fori_loop(0, (NCH + NBS - 1) // NBS, body, 0)
        for b in range(NBS):
            pltpu.make_async_copy(buf.at[b], accum.at[idx_d.at[0]],
                                  ss[b]).wait()
        plsc.subcore_barrier()
        pltpu.sync_copy(accum.at[pl.ds(r0, NPT)],
                        out_hbm.at[cid, pl.ds(r0, NPT)])

    return _sc_gather2, _sc_scatter


def _dot(a, b):
    return jnp.dot(a, b, preferred_element_type=_F32)


def _dotb(a, b):
    return jnp.dot(a.astype(_BF16), b.astype(_BF16),
                   preferred_element_type=_F32)


def _ln_block(m, g, b):
    mu = jnp.mean(m, axis=-1, keepdims=True)
    d = m - mu
    var = jnp.mean(d * d, axis=-1, keepdims=True)
    return d * lax.rsqrt(var + 1e-5) * g + b


def _prep_body(x, wa, wb, po, qo):
    xv = x[...]
    po[...] = _dot(xv, wa[...])
    qo[...] = _dot(xv, wb[...])


def _edge_body(pg, qg, ea, w1c, b1, w2, b2, w3, b3, w4, b4, g, bb,
               msg_o, ean_o):
    ea_v = ea[...]
    h = pg[...] + qg[...] + _dotb(ea_v, w1c[...]) + b1[...]
    h = jax.nn.relu(h)
    h = jax.nn.relu(_dotb(h, w2[...]) + b2[...])
    h = jax.nn.relu(_dotb(h, w3[...]) + b3[...])
    m = _dotb(h, w4[...]) + b4[...]
    msg = _ln_block(m, g[...], bb[...])
    msg_o[...] = msg
    ean_o[...] = ea_v + msg


def _node_body(x, p0, p1, w1a, w1b, b1, w2, b2, w3, b3, w4, b4,
               g, bb, *rest):
    xv = x[...]
    agg = p0[...] + p1[...]
    h = _dotb(xv, w1a[...]) + _dotb(agg, w1b[...]) + b1[...]
    h = jax.nn.relu(h)
    h = jax.nn.relu(_dotb(h, w2[...]) + b2[...])
    h = jax.nn.relu(_dotb(h, w3[...]) + b3[...])
    u = _dotb(h, w4[...]) + b4[...]
    xn = xv + _ln_block(u, g[...], bb[...])
    if len(rest) == 1:
        (xo,) = rest
        xo[...] = xn
    else:
        wea, web, xo, po, qo = rest
        xo[...] = xn
        po[...] = _dot(xn, wea[...])
        qo[...] = _dot(xn, web[...])


_EB = 8000
_NB = 2000


def _bspec(rows):
    return pl.BlockSpec((rows, L), lambda i: (i, 0))


_WSPEC = pl.BlockSpec((L, L), lambda i: (0, 0))
_VSPEC = pl.BlockSpec((1, L), lambda i: (0, 0))


def _prep_tc(x, wa, wb):
    return pl.pallas_call(
        _prep_body,
        grid=(NN // _NB,),
        in_specs=[_bspec(_NB), _WSPEC, _WSPEC],
        out_specs=[_bspec(_NB), _bspec(_NB)],
        out_shape=[jax.ShapeDtypeStruct((NN, L), _F32)] * 2,
    )(x, wa, wb)


def _edge_tc(pg, qg, ea, w1c, b1, w2, b2, w3, b3, w4, b4, g, bb):
    blk = _bspec(_EB)
    return pl.pallas_call(
        _edge_body,
        grid=(NE // _EB,),
        in_specs=[blk, blk, blk, _WSPEC, _VSPEC, _WSPEC, _VSPEC,
                  _WSPEC, _VSPEC, _WSPEC, _VSPEC, _VSPEC, _VSPEC],
        out_specs=[blk, blk],
        out_shape=[jax.ShapeDtypeStruct((NE, L), _F32)] * 2,
    )(pg, qg, ea, w1c, b1, w2, b2, w3, b3, w4, b4, g, bb)


def _node_tc(x, parts, w1a, w1b, b1, w2, b2, w3, b3, w4, b4, g, bb,
             wea=None, web=None):
    blk = _bspec(_NB)
    n_out = 1 if wea is None else 3
    specs = [blk] * 3 + [_WSPEC, _WSPEC, _VSPEC, _WSPEC, _VSPEC,
                         _WSPEC, _VSPEC, _WSPEC, _VSPEC, _VSPEC, _VSPEC]
    args = [x] + parts + [w1a, w1b, b1, w2, b2, w3, b3, w4, b4, g, bb]
    if wea is not None:
        specs += [_WSPEC, _WSPEC]
        args += [wea, web]
    return pl.pallas_call(
        _node_body,
        grid=(NN // _NB,),
        in_specs=specs,
        out_specs=[blk] * n_out,
        out_shape=[jax.ShapeDtypeStruct((NN, L), _F32)] * n_out,
    )(*args)


def kernel(x, edge_attr, edge_index, We1, be1, We2, be2, We3, be3, We4, be4,
           ge, bbe, Wn1, bn1, Wn2, bn2, Wn3, bn3, Wn4, bn4, gn, bbn):
    src = edge_index[0]
    dst = edge_index[1]
    di3 = dst.reshape(NW, NCH, CHUNK)
    si3 = src.reshape(NW, NCH, CHUNK)
    zeros = jnp.zeros((NNP, L), _F32)

    ea = edge_attr
    sc_gather2, sc_scatter = _sc_kernels()
    P, Q = _prep_tc(x, We1[0, :L], We1[0, L:2 * L])
    for s in range(2):
        ew = (We1[s, 2 * L:], be1[s][None], We2[s], be2[s][None],
              We3[s], be3[s][None], We4[s], be4[s][None],
              ge[s][None], bbe[s][None])
        pg, qg = sc_gather2(P, Q, di3, si3)
        msg, ea = _edge_tc(pg, qg, ea, *ew)
        pk = sc_scatter(msg, di3, zeros)
        nw = (Wn1[s, :L], Wn1[s, L:], bn1[s][None], Wn2[s], bn2[s][None],
              Wn3[s], bn3[s][None], Wn4[s], bn4[s][None],
              gn[s][None], bbn[s][None])
        if s == 0:
            x, P, Q = _node_tc(x, [pk[0], pk[1]], *nw,
                               wea=We1[1, :L], web=We1[1, L:2 * L])
        else:
            (x,) = _node_tc(x, [pk[0], pk[1]], *nw)
    return (x, ea)
